# baseline (speedup 1.0000x reference)
import functools
from typing import Any

import jax
import jax.numpy as jnp
from jax import lax
from jax.experimental import pallas as pl
from jax.experimental.pallas import tpu as pltpu

N0, N1 = 2000, 8000
N = N0 + N1
D_HID, H, AV = 64, 8, 128
E = 160000
IN0, IN1, EMB_D, OUT_D = 128, 64, 64, 8

NEG_BIG = -9e15


def _h1_matmul_kernel(src_ref, w_ref, out_ref):
    out_ref[0] = src_ref[:] @ w_ref[0]


def _compute_h1_all(emb_src, W_ac):
    C = emb_src.shape[0]
    return pl.pallas_call(
        _h1_matmul_kernel,
        grid=(H,),
        in_specs=[
            pl.BlockSpec((C, EMB_D), lambda h: (0, 0)),
            pl.BlockSpec((1, EMB_D, AV), lambda h: (h, 0, 0)),
        ],
        out_specs=pl.BlockSpec((1, C, AV), lambda h: (h, 0, 0)),
        out_shape=jax.ShapeDtypeStruct((H, C, AV), jnp.float32),
    )(emb_src, W_ac)


def _hgnn_attn_kernel(dest_ref, w_ref, h1_ref, bias_ref, keep_ref, fsrc_ref,
                      out_ref):
    h = pl.program_id(1)
    h2 = dest_ref[:] @ w_ref[0]
    e = lax.dot_general(h2, h1_ref[0], (((1,), (1,)), ((), ())))
    e = jnp.where(e > 0, e, jnp.exp(e) - 1.0)
    adj_ok = bias_ref[:] > 0
    keep_ok = keep_ref[:] > 0.5
    logits = jnp.where(keep_ok, jnp.where(adj_ok, e, NEG_BIG), -jnp.inf)
    m = jnp.max(logits, axis=1, keepdims=True)
    p = jnp.exp(logits - m)
    s = jnp.sum(p, axis=1, keepdims=True)
    att = p / s
    contrib = att @ fsrc_ref[:]

    @pl.when(h == 0)
    def _():
        out_ref[:] = jnp.zeros_like(out_ref)

    out_ref[:] += contrib * (1.0 / H)


def _hgnn_ac_fused(bias_full, emb_dest, emb_src_full, fsrc_full, keep_mask,
                   W_ac):
    R, C = bias_full.shape
    BR = 200
    h1_all = _compute_h1_all(emb_src_full, W_ac)
    return pl.pallas_call(
        _hgnn_attn_kernel,
        grid=(R // BR, H),
        in_specs=[
            pl.BlockSpec((BR, EMB_D), lambda r, h: (r, 0)),
            pl.BlockSpec((1, EMB_D, AV), lambda r, h: (h, 0, 0)),
            pl.BlockSpec((1, C, AV), lambda r, h: (h, 0, 0)),
            pl.BlockSpec((BR, C), lambda r, h: (r, 0)),
            pl.BlockSpec((1, C), lambda r, h: (0, 0)),
            pl.BlockSpec((C, D_HID), lambda r, h: (0, 0)),
        ],
        out_specs=pl.BlockSpec((BR, D_HID), lambda r, h: (r, 0)),
        out_shape=jax.ShapeDtypeStruct((R, D_HID), jnp.float32),
        compiler_params=pltpu.CompilerParams(
            dimension_semantics=("arbitrary", "arbitrary")),
    )(emb_dest, W_ac, h1_all, bias_full, keep_mask[None, :], fsrc_full)


def _seg_softmax(a, seg, n):
    m = jax.ops.segment_max(a, seg, num_segments=n)
    m = jnp.where(jnp.isfinite(m), m, 0.0)
    e = jnp.exp(a - m[seg])
    s = jax.ops.segment_sum(e, seg, num_segments=n)
    return e / (s[seg] + 1e-9)


def _metapath(features, emi, offset, n_dst, attn):
    edata = features[emi]
    hidden = edata.mean(axis=1)
    a = jax.nn.leaky_relu(hidden @ attn.T, 0.01)
    dst = emi[:, -1] - offset
    att = _seg_softmax(a, dst, n_dst)
    outs = [jax.ops.segment_sum(hidden * att[:, h:h + 1], dst,
                                num_segments=n_dst) for h in range(H)]
    out = jnp.stack(outs, axis=1)
    return jax.nn.elu(out.reshape(n_dst, H * D_HID))


def _ctr(features, emis, offset, n_dst, attns, w1, b1, w2):
    outs = [_metapath(features, emis[p], offset, n_dst, attns[p])
            for p in range(len(emis))]
    scores = jnp.stack([jnp.tanh(o @ w1 + b1).mean(axis=0) @ w2 for o in outs])
    beta = jax.nn.softmax(scores)
    h = beta[0] * outs[0]
    for p in range(1, len(outs)):
        h = h + beta[p] * outs[p]
    return h, beta


def _layer(features, emis0, emis1, attn, s1W, s1b, s2, fcW, fcb):
    h0, _ = _ctr(features, emis0, 0, N0, attn[0], s1W[0], s1b[0], s2[0])
    h1, beta = _ctr(features, emis1, N0, N1, attn[1], s1W[1], s1b[1], s2[1])
    h = jnp.concatenate([h0, h1], axis=0)
    return h @ fcW + fcb, h, beta


def kernel(feat0, feat1, emb, adj, type_mask, feat_keep_idx, feat_drop_idx,
           emi00, emi01, emi10, emi11, target_node_indices,
           fc0_W, fc0_b, fc1_W, fc1_b, W_ac, attn_l0, attn_l1,
           sem_fc1W_l0, sem_fc1b_l0, sem_fc2_l0,
           sem_fc1W_l1, sem_fc1b_l1, sem_fc2_l1,
           fcW_l0, fcb_l0, fcW_l1, fcb_l1):
    tf0 = feat0 @ fc0_W + fc0_b
    tf1 = feat1 @ fc1_W + fc1_b
    feat_src = tf0

    keep_mask = jnp.zeros((N0,), jnp.float32).at[feat_keep_idx].set(1.0)

    feat_src_re = _hgnn_ac_fused(adj[:N0], emb[:N0], emb[:N0], feat_src,
                                 keep_mask, W_ac)
    a = feat_src[feat_drop_idx]
    b = feat_src_re[feat_drop_idx]
    cos = jnp.sum(a * b, axis=1) / jnp.maximum(
        jnp.linalg.norm(a, axis=1) * jnp.linalg.norm(b, axis=1), 1e-8)
    loss_ac = 1.0 - jnp.sum(cos) / a.shape[0]

    ones_mask = jnp.ones((N0,), jnp.float32)
    feat_ac1 = _hgnn_ac_fused(adj[N0:], emb[N0:], emb[:N0], feat_src,
                              ones_mask, W_ac)
    h = jnp.concatenate([tf0, feat_ac1], axis=0)

    emis0 = [emi00, emi01]
    emis1 = [emi10, emi11]
    h_fc, _, _ = _layer(h, emis0, emis1, attn_l0, sem_fc1W_l0, sem_fc1b_l0,
                        sem_fc2_l0, fcW_l0, fcb_l0)
    h = jax.nn.elu(h_fc)
    logits, h_last, beta = _layer(h, emis0, emis1, attn_l1, sem_fc1W_l1,
                                  sem_fc1b_l1, sem_fc2_l1, fcW_l1, fcb_l1)
    return (logits[target_node_indices], h_last[target_node_indices], beta,
            loss_ac)
